# baseline (speedup 1.0000x reference)
import functools

import jax
import jax.numpy as jnp
from jax import lax
from jax.experimental import pallas as pl
from jax.experimental.pallas import tpu as pltpu
from jax.experimental.pallas import tpu_sc as plsc

N, E, D, H, ED = 10000, 320000, 128, 128, 16
NC, NS = 2, 16
NW = NC * NS
EW = E // NW
C = 80
NCHUNK = EW // C
ROWS_PER_TILE = N // NS

EB = 2048
SLAB = 163840
EFULL = 2 * SLAB
GRID = SLAB // EB
GEW = SLAB // NW
GNCHUNK = GEW // C
S0EW = SLAB // NW
S0CH = S0EW // C
E1R = E - SLAB
S1EW = E1R // NW
S1CH = S1EW // C
S1CH2 = S1EW // C2 if False else None

_mesh = plsc.VectorSubcoreMesh(core_axis_name="c", subcore_axis_name="s")


@functools.partial(
    pl.kernel,
    out_type=(
        jax.ShapeDtypeStruct((SLAB, D), jnp.float32),
        jax.ShapeDtypeStruct((SLAB, D), jnp.float32),
    ),
    mesh=_mesh,
    compiler_params=pltpu.CompilerParams(use_tc_tiling_on_sc=False),
    scratch_types=[
        pltpu.VMEM((C,), jnp.int32),
        pltpu.VMEM((C,), jnp.int32),
        pltpu.VMEM((C, D), jnp.float32),
        pltpu.VMEM((C, D), jnp.float32),
        pltpu.SemaphoreType.DMA,
        pltpu.SemaphoreType.DMA,
    ],
)
def _gather_k(h_hbm, src_hbm, dst_hbm, gs_hbm, gd_hbm,
              idx_s, idx_d, buf_s, buf_d, sem_s, sem_d):
    wid = lax.axis_index("s") * NC + lax.axis_index("c")
    base = wid * GEW

    def body(i, carry):
        off = base + i * C
        pltpu.sync_copy(src_hbm.at[pl.ds(off, C)], idx_s)
        pltpu.sync_copy(dst_hbm.at[pl.ds(off, C)], idx_d)
        cp_s = pltpu.async_copy(h_hbm.at[idx_s], buf_s, sem_s)
        cp_d = pltpu.async_copy(h_hbm.at[idx_d], buf_d, sem_d)
        cp_s.wait()
        cp_d.wait()
        pltpu.sync_copy(buf_s, gs_hbm.at[pl.ds(off, C)])
        pltpu.sync_copy(buf_d, gd_hbm.at[pl.ds(off, C)])
        return carry

    lax.fori_loop(0, GNCHUNK, body, 0)


def _mlp_body(gs, gd, dist8, w1e, b1e, w2e, b2e,
              w1na, w1nb, w1ne, b1n, w2n, b2n,
              w1ca, w1cb, w1ce, b1c, w2ct,
              m_out, wt_out):
    bf = jnp.bfloat16
    f32 = jnp.float32
    nsub = EB // 128
    sel = (lax.broadcasted_iota(jnp.int32, (EB, nsub), 0) // 128
           == lax.broadcasted_iota(jnp.int32, (EB, nsub), 1)).astype(f32)
    xrows = jnp.dot(sel, dist8[...], preferred_element_type=f32)
    lmask = (lax.broadcasted_iota(jnp.int32, (EB, 128), 1)
             == lax.broadcasted_iota(jnp.int32, (EB, 128), 0) % 128)
    d = jnp.sum(jnp.where(lmask, xrows, 0.0), axis=1, keepdims=True)

    a1 = d * w1e[...] + b1e[...]
    a1 = a1 * jax.nn.sigmoid(a1)
    attr = jnp.dot(a1, w2e[...], preferred_element_type=f32) + b2e[...]
    hs = gs[...].astype(bf)
    hd = gd[...].astype(bf)
    pre_n = (jnp.dot(hs, w1na[...].astype(bf), preferred_element_type=f32)
             + jnp.dot(hd, w1nb[...].astype(bf), preferred_element_type=f32)
             + jnp.dot(attr.astype(bf), w1ne[...].astype(bf),
                       preferred_element_type=f32)
             + b1n[...])
    hid_n = pre_n * jax.nn.sigmoid(pre_n)
    m_out[...] = jnp.dot(hid_n.astype(bf), w2n[...].astype(bf),
                         preferred_element_type=f32) + b2n[...]
    pre_c = (jnp.dot(hs, w1ca[...].astype(bf), preferred_element_type=f32)
             + jnp.dot(hd, w1cb[...].astype(bf), preferred_element_type=f32)
             + jnp.dot(attr.astype(bf), w1ce[...].astype(bf),
                       preferred_element_type=f32)
             + b1c[...])
    hid_c = pre_c * jax.nn.sigmoid(pre_c)
    wrow = lax.dot_general(w2ct[...], hid_c,
                           (((1,), (1,)), ((), ())),
                           preferred_element_type=f32)
    wt_out[...] = wrow.reshape(1, 1, EB)


def _full(shape):
    return pl.BlockSpec(shape, lambda i: (0, 0))


_mlp_call = pl.pallas_call(
    _mlp_body,
    grid=(GRID,),
    in_specs=[
        pl.BlockSpec((EB, D), lambda i: (i, 0)),
        pl.BlockSpec((EB, D), lambda i: (i, 0)),
        pl.BlockSpec((EB // 128, 128), lambda i: (i, 0)),
        _full((1, ED)), _full((1, ED)), _full((ED, ED)), _full((1, ED)),
        _full((D, H)), _full((D, H)), _full((ED, H)), _full((1, H)),
        _full((H, D)), _full((1, D)),
        _full((D, H)), _full((D, H)), _full((ED, H)), _full((1, H)),
        _full((1, H)),
    ],
    out_specs=[
        pl.BlockSpec((EB, D), lambda i: (i, 0)),
        pl.BlockSpec((1, 1, EB), lambda i: (i, 0, 0)),
    ],
    out_shape=[
        jax.ShapeDtypeStruct((SLAB, D), jnp.float32),
        jax.ShapeDtypeStruct((GRID, 1, EB), jnp.float32),
    ],
)


@functools.partial(
    pl.kernel,
    out_type=jax.ShapeDtypeStruct((NC, N, D), jnp.float32),
    mesh=_mesh,
    compiler_params=pltpu.CompilerParams(use_tc_tiling_on_sc=False),
    scratch_types=[
        pltpu.VMEM((C,), jnp.int32),
        pltpu.VMEM((C, D), jnp.float32),
        pltpu.VMEM_SHARED((N, D), jnp.float32),
    ],
)
def _scatter_m_k(m0_hbm, m1_hbm, dst0_hbm, dst1_hbm, z128_hbm, hp_hbm,
                 ixd0, mb0, h_acc):
    cid = lax.axis_index("c")
    sid = lax.axis_index("s")
    wid = sid * NC + cid
    r0 = sid * ROWS_PER_TILE
    pltpu.sync_copy(z128_hbm.at[pl.ds(r0, ROWS_PER_TILE)],
                    h_acc.at[pl.ds(r0, ROWS_PER_TILE)])
    plsc.subcore_barrier()
    base0 = wid * S0EW
    base1 = wid * S1EW

    def body0(i, carry):
        off = base0 + i * C
        pltpu.sync_copy(dst0_hbm.at[pl.ds(off, C)], ixd0)
        pltpu.sync_copy(m0_hbm.at[pl.ds(off, C)], mb0)
        pltpu.sync_copy(mb0, h_acc.at[ixd0], add=True)
        return carry

    def body1(i, carry):
        off = base1 + i * C
        pltpu.sync_copy(dst1_hbm.at[pl.ds(off, C)], ixd0)
        pltpu.sync_copy(m1_hbm.at[pl.ds(off, C)], mb0)
        pltpu.sync_copy(mb0, h_acc.at[ixd0], add=True)
        return carry

    lax.fori_loop(0, S0CH, body0, 0)
    lax.fori_loop(0, S1CH, body1, 0)
    plsc.subcore_barrier()
    pltpu.sync_copy(h_acc.at[pl.ds(r0, ROWS_PER_TILE)],
                    hp_hbm.at[cid, pl.ds(r0, ROWS_PER_TILE)])


C2 = 80


@functools.partial(
    pl.kernel,
    out_type=jax.ShapeDtypeStruct((NW, N * 4), jnp.float32),
    mesh=_mesh,
    compiler_params=pltpu.CompilerParams(use_tc_tiling_on_sc=False,
                                         needs_layout_passes=False),
    scratch_types=[
        pltpu.VMEM((C2,), jnp.int32),
        pltpu.VMEM((C2,), jnp.int32),
        pltpu.VMEM((C2,), jnp.float32),
        pltpu.VMEM((N * 4,), jnp.float32),
        pltpu.VMEM((N * 4,), jnp.float32),
    ],
)
def _scatter_x_k(w0_hbm, w1_hbm, src0_hbm, src1_hbm, dst0_hbm, dst1_hbm,
                 x4_hbm, z4_hbm, xp_hbm,
                 idx_s, idx_d, w_buf, xtab, xacc):
    cid = lax.axis_index("c")
    sid = lax.axis_index("s")
    wid = sid * NC + cid
    pltpu.sync_copy(x4_hbm, xtab)
    pltpu.sync_copy(z4_hbm, xacc)

    def make_body(w_hbm, src_hbm, dst_hbm, base):
        def body(i, carry):
            off = base + i * C2
            pltpu.sync_copy(src_hbm.at[pl.ds(off, C2)], idx_s)
            pltpu.sync_copy(dst_hbm.at[pl.ds(off, C2)], idx_d)
            pltpu.sync_copy(w_hbm.at[pl.ds(off, C2)], w_buf)
            for j in range(C2 // 16):
                s16 = idx_s[pl.ds(j * 16, 16)] * 4
                d16 = idx_d[pl.ds(j * 16, 16)] * 4
                w16 = w_buf[pl.ds(j * 16, 16)]
                for comp in range(3):
                    xs = plsc.load_gather(xtab, [s16 + comp])
                    xd = plsc.load_gather(xtab, [d16 + comp])
                    plsc.addupdate_scatter(xacc, [d16 + comp], w16 * (xs - xd))
            return carry
        return body

    lax.fori_loop(0, S0EW // C2, make_body(w0_hbm, src0_hbm, dst0_hbm,
                                           wid * S0EW), 0)
    lax.fori_loop(0, S1EW // C2, make_body(w1_hbm, src1_hbm, dst1_hbm,
                                           wid * S1EW), 0)
    pltpu.sync_copy(xacc, xp_hbm.at[wid])


def kernel(h, x, edge_index, edge_dist,
           W1e, b1e, W2e, b2e, W1n, b1n, W2n, b2n, W1c, b1c, W2c):
    src = edge_index[0]
    dst = edge_index[1]
    srcp = jnp.pad(src, (0, EFULL - E))
    dstp = jnp.pad(dst, (0, EFULL - E))
    src0, src1 = srcp[:SLAB], srcp[SLAB:]
    dst0, dst1 = dstp[:SLAB], dstp[SLAB:]
    distp = jnp.pad(edge_dist, (0, EFULL - E)).reshape(2, SLAB // 128, 128)
    gs0, gd0 = _gather_k(h, src0, dst0)
    gs1, gd1 = _gather_k(h, src1, dst1)
    wargs = (W1e, b1e.reshape(1, ED), W2e, b2e.reshape(1, ED),
             W1n[:D], W1n[D:2 * D], W1n[2 * D:], b1n.reshape(1, H),
             W2n, b2n.reshape(1, D),
             W1c[:D], W1c[D:2 * D], W1c[2 * D:], b1c.reshape(1, H),
             W2c.reshape(1, H))
    m0, wt0 = _mlp_call(gs0, gd0, distp[0], *wargs)
    m1, wt1 = _mlp_call(gs1, gd1, distp[1], *wargs)
    w0 = wt0.reshape(SLAB)
    w1 = wt1.reshape(SLAB)
    x4 = jnp.pad(x, ((0, 0), (0, 1))).reshape(N * 4)
    z128 = jnp.zeros((N, D), jnp.float32)
    z4 = jnp.zeros((N * 4,), jnp.float32)
    hp = _scatter_m_k(m0, m1, dst0, dst1, z128)
    xp = _scatter_x_k(w0, w1, src0, src1, dst0, dst1, x4, z4)
    h_out = h + hp[0] + hp[1]
    x_out = x + jnp.sum(xp, axis=0).reshape(N, 4)[:, :3]
    return (h_out, x_out)


# speedup vs baseline: 1.5570x; 1.5570x over previous
import functools

import jax
import jax.numpy as jnp
from jax import lax
from jax.experimental import pallas as pl
from jax.experimental.pallas import tpu as pltpu
from jax.experimental.pallas import tpu_sc as plsc

N, E, D, H, ED = 10000, 320000, 128, 128, 16
NC, NS = 2, 16
NW = NC * NS
EW = E // NW
C = 80
NCHUNK = EW // C
ROWS_PER_TILE = N // NS

EB = 2048
EPAD = 321536
GRID = EPAD // EB

_mesh = plsc.VectorSubcoreMesh(core_axis_name="c", subcore_axis_name="s")


@functools.partial(
    pl.kernel,
    out_type=(
        jax.ShapeDtypeStruct((EPAD, D), jnp.float32),
        jax.ShapeDtypeStruct((EPAD, D), jnp.float32),
    ),
    mesh=_mesh,
    compiler_params=pltpu.CompilerParams(use_tc_tiling_on_sc=False),
    scratch_types=[
        pltpu.VMEM((C,), jnp.int32),
        pltpu.VMEM((C,), jnp.int32),
        pltpu.VMEM((C, D), jnp.float32),
        pltpu.VMEM((C, D), jnp.float32),
        pltpu.SemaphoreType.DMA,
        pltpu.SemaphoreType.DMA,
    ],
)
def _gather_k(h_hbm, src_hbm, dst_hbm, gs_hbm, gd_hbm,
              idx_s, idx_d, buf_s, buf_d, sem_s, sem_d):
    wid = lax.axis_index("s") * NC + lax.axis_index("c")
    base = wid * EW

    def body(i, carry):
        off = base + i * C
        pltpu.sync_copy(src_hbm.at[pl.ds(off, C)], idx_s)
        pltpu.sync_copy(dst_hbm.at[pl.ds(off, C)], idx_d)
        cp_s = pltpu.async_copy(h_hbm.at[idx_s], buf_s, sem_s)
        cp_d = pltpu.async_copy(h_hbm.at[idx_d], buf_d, sem_d)
        cp_s.wait()
        cp_d.wait()
        pltpu.sync_copy(buf_s, gs_hbm.at[pl.ds(off, C)])
        pltpu.sync_copy(buf_d, gd_hbm.at[pl.ds(off, C)])
        return carry

    lax.fori_loop(0, NCHUNK, body, 0)


def _mlp_body(gs, gd, dist8, w1e, b1e, w2e, b2e,
              w1na, w1nb, w1ne, b1n, w2n, b2n,
              w1ca, w1cb, w1ce, b1c, w2ct,
              m_out, wt_out):
    bf = jnp.bfloat16
    f32 = jnp.float32
    nsub = EB // 128
    sel = (lax.broadcasted_iota(jnp.int32, (EB, nsub), 0) // 128
           == lax.broadcasted_iota(jnp.int32, (EB, nsub), 1)).astype(f32)
    xrows = jnp.dot(sel, dist8[...], preferred_element_type=f32)
    lmask = (lax.broadcasted_iota(jnp.int32, (EB, 128), 1)
             == lax.broadcasted_iota(jnp.int32, (EB, 128), 0) % 128)
    d = jnp.sum(jnp.where(lmask, xrows, 0.0), axis=1, keepdims=True)

    a1 = d * w1e[...] + b1e[...]
    a1 = a1 * jax.nn.sigmoid(a1)
    attr = jnp.dot(a1, w2e[...], preferred_element_type=f32) + b2e[...]
    hs = gs[...].astype(bf)
    hd = gd[...].astype(bf)
    pre_n = (jnp.dot(hs, w1na[...].astype(bf), preferred_element_type=f32)
             + jnp.dot(hd, w1nb[...].astype(bf), preferred_element_type=f32)
             + jnp.dot(attr.astype(bf), w1ne[...].astype(bf),
                       preferred_element_type=f32)
             + b1n[...])
    hid_n = pre_n * jax.nn.sigmoid(pre_n)
    m_out[...] = jnp.dot(hid_n.astype(bf), w2n[...].astype(bf),
                         preferred_element_type=f32) + b2n[...]
    pre_c = (jnp.dot(hs, w1ca[...].astype(bf), preferred_element_type=f32)
             + jnp.dot(hd, w1cb[...].astype(bf), preferred_element_type=f32)
             + jnp.dot(attr.astype(bf), w1ce[...].astype(bf),
                       preferred_element_type=f32)
             + b1c[...])
    hid_c = pre_c * jax.nn.sigmoid(pre_c)
    wrow = lax.dot_general(w2ct[...], hid_c,
                           (((1,), (1,)), ((), ())),
                           preferred_element_type=f32)
    wt_out[...] = wrow.reshape(1, 1, EB)


def _full(shape):
    return pl.BlockSpec(shape, lambda i: (0, 0))


_mlp_call = pl.pallas_call(
    _mlp_body,
    grid=(GRID,),
    in_specs=[
        pl.BlockSpec((EB, D), lambda i: (i, 0)),
        pl.BlockSpec((EB, D), lambda i: (i, 0)),
        pl.BlockSpec((EB // 128, 128), lambda i: (i, 0)),
        _full((1, ED)), _full((1, ED)), _full((ED, ED)), _full((1, ED)),
        _full((D, H)), _full((D, H)), _full((ED, H)), _full((1, H)),
        _full((H, D)), _full((1, D)),
        _full((D, H)), _full((D, H)), _full((ED, H)), _full((1, H)),
        _full((1, H)),
    ],
    out_specs=[
        pl.BlockSpec((EB, D), lambda i: (i, 0)),
        pl.BlockSpec((1, 1, EB), lambda i: (i, 0, 0)),
    ],
    out_shape=[
        jax.ShapeDtypeStruct((EPAD, D), jnp.float32),
        jax.ShapeDtypeStruct((GRID, 1, EB), jnp.float32),
    ],
)


@functools.partial(
    pl.kernel,
    out_type=jax.ShapeDtypeStruct((NC, N, D), jnp.float32),
    mesh=_mesh,
    compiler_params=pltpu.CompilerParams(use_tc_tiling_on_sc=False),
    scratch_types=[
        pltpu.VMEM((C,), jnp.int32),
        pltpu.VMEM((C, D), jnp.float32),
        pltpu.VMEM_SHARED((N, D), jnp.float32),
    ],
)
def _scatter_m_k(m_hbm, dst_hbm, z128_hbm, hp_hbm, ixd0, mb0, h_acc):
    cid = lax.axis_index("c")
    sid = lax.axis_index("s")
    wid = sid * NC + cid
    r0 = sid * ROWS_PER_TILE
    pltpu.sync_copy(z128_hbm.at[pl.ds(r0, ROWS_PER_TILE)],
                    h_acc.at[pl.ds(r0, ROWS_PER_TILE)])
    plsc.subcore_barrier()
    base = wid * EW

    def body(i, carry):
        off = base + i * C
        pltpu.sync_copy(dst_hbm.at[pl.ds(off, C)], ixd0)
        pltpu.sync_copy(m_hbm.at[pl.ds(off, C)], mb0)
        pltpu.sync_copy(mb0, h_acc.at[ixd0], add=True)
        return carry

    lax.fori_loop(0, NCHUNK, body, 0)
    plsc.subcore_barrier()
    pltpu.sync_copy(h_acc.at[pl.ds(r0, ROWS_PER_TILE)],
                    hp_hbm.at[cid, pl.ds(r0, ROWS_PER_TILE)])


C2 = 400
NCHUNK2 = EW // C2


@functools.partial(
    pl.kernel,
    out_type=jax.ShapeDtypeStruct((NW, N * 4), jnp.float32),
    mesh=_mesh,
    compiler_params=pltpu.CompilerParams(use_tc_tiling_on_sc=False,
                                         needs_layout_passes=False),
    scratch_types=[
        pltpu.VMEM((C2,), jnp.int32),
        pltpu.VMEM((C2,), jnp.int32),
        pltpu.VMEM((C2,), jnp.float32),
        pltpu.VMEM((N * 4,), jnp.float32),
        pltpu.VMEM((N * 4,), jnp.float32),
    ],
)
def _scatter_x_k(w_hbm, src_hbm, dst_hbm, x4_hbm, z4_hbm, xp_hbm,
                 idx_s, idx_d, w_buf, xtab, xacc):
    cid = lax.axis_index("c")
    sid = lax.axis_index("s")
    wid = sid * NC + cid
    pltpu.sync_copy(x4_hbm, xtab)
    pltpu.sync_copy(z4_hbm, xacc)
    base = wid * EW
    lane = lax.iota(jnp.int32, 16)

    def body(i, carry):
        off = base + i * C2
        pltpu.sync_copy(src_hbm.at[pl.ds(off, C2)], idx_s)
        pltpu.sync_copy(dst_hbm.at[pl.ds(off, C2)], idx_d)
        pltpu.sync_copy(w_hbm.at[pl.ds(off, C2)], w_buf)
        for j in range(C2 // 16):
            s16 = idx_s[pl.ds(j * 16, 16)] * 4
            d16 = idx_d[pl.ds(j * 16, 16)] * 4
            w16 = w_buf[pl.ds(j * 16, 16)]
            for comp in range(3):
                xs = plsc.load_gather(xtab, [s16 + comp])
                xd = plsc.load_gather(xtab, [d16 + comp])
                plsc.addupdate_scatter(xacc, [d16 + comp], w16 * (xs - xd))
        return carry

    lax.fori_loop(0, NCHUNK2, body, 0)
    pltpu.sync_copy(xacc, xp_hbm.at[wid])


def kernel(h, x, edge_index, edge_dist,
           W1e, b1e, W2e, b2e, W1n, b1n, W2n, b2n, W1c, b1c, W2c):
    src = edge_index[0]
    dst = edge_index[1]
    gs, gd = _gather_k(h, src, dst)
    dist_pad = jnp.pad(edge_dist, (0, EPAD - E)).reshape(EPAD // 128, 128)
    m, wt = _mlp_call(
        gs, gd, dist_pad,
        W1e, b1e.reshape(1, ED), W2e, b2e.reshape(1, ED),
        W1n[:D], W1n[D:2 * D], W1n[2 * D:], b1n.reshape(1, H),
        W2n, b2n.reshape(1, D),
        W1c[:D], W1c[D:2 * D], W1c[2 * D:], b1c.reshape(1, H),
        W2c.reshape(1, H),
    )
    w1d = wt.reshape(EPAD)
    x4 = jnp.pad(x, ((0, 0), (0, 1))).reshape(N * 4)
    z128 = jnp.zeros((N, D), jnp.float32)
    z4 = jnp.zeros((N * 4,), jnp.float32)
    hp = _scatter_m_k(m, dst, z128)
    xp = _scatter_x_k(w1d, src, dst, x4, z4)
    h_out = h + hp[0] + hp[1]
    x_out = x + jnp.sum(xp, axis=0).reshape(N, 4)[:, :3]
    return (h_out, x_out)


# speedup vs baseline: 1.7266x; 1.1089x over previous
import functools

import jax
import jax.numpy as jnp
from jax import lax
from jax.experimental import pallas as pl
from jax.experimental.pallas import tpu as pltpu
from jax.experimental.pallas import tpu_sc as plsc

N, E, D, H, ED = 10000, 320000, 128, 128, 16
NC, NS = 2, 16
NW = NC * NS
EW = E // NW
C = 80
NCHUNK = EW // C
ROWS_PER_TILE = N // NS

EB = 2048
EPAD = 321536
GRID = EPAD // EB

_mesh = plsc.VectorSubcoreMesh(core_axis_name="c", subcore_axis_name="s")


CG = 200
NCHG = EW // CG


@functools.partial(
    pl.kernel,
    out_type=(
        jax.ShapeDtypeStruct((EPAD, D), jnp.float32),
        jax.ShapeDtypeStruct((EPAD, D), jnp.float32),
    ),
    mesh=_mesh,
    compiler_params=pltpu.CompilerParams(use_tc_tiling_on_sc=False),
    scratch_types=[
        pltpu.VMEM((CG,), jnp.int32),
        pltpu.VMEM((CG,), jnp.int32),
        pltpu.VMEM((CG, D), jnp.float32),
        pltpu.VMEM((CG, D), jnp.float32),
        pltpu.SemaphoreType.DMA,
        pltpu.SemaphoreType.DMA,
    ],
)
def _gather_k(h_hbm, src_hbm, dst_hbm, gs_hbm, gd_hbm,
              idx_s, idx_d, buf_s, buf_d, sem_s, sem_d):
    wid = lax.axis_index("s") * NC + lax.axis_index("c")
    base = wid * EW

    def body(i, carry):
        off = base + i * CG
        pltpu.sync_copy(src_hbm.at[pl.ds(off, CG)], idx_s)
        pltpu.sync_copy(dst_hbm.at[pl.ds(off, CG)], idx_d)
        cs0 = pltpu.async_copy(h_hbm.at[idx_s.at[pl.ds(0, 128)]],
                               buf_s.at[pl.ds(0, 128)], sem_s)
        cs1 = pltpu.async_copy(h_hbm.at[idx_s.at[pl.ds(128, CG - 128)]],
                               buf_s.at[pl.ds(128, CG - 128)], sem_s)
        cd0 = pltpu.async_copy(h_hbm.at[idx_d.at[pl.ds(0, 128)]],
                               buf_d.at[pl.ds(0, 128)], sem_d)
        cd1 = pltpu.async_copy(h_hbm.at[idx_d.at[pl.ds(128, CG - 128)]],
                               buf_d.at[pl.ds(128, CG - 128)], sem_d)
        cs0.wait()
        cs1.wait()
        pltpu.sync_copy(buf_s, gs_hbm.at[pl.ds(off, CG)])
        cd0.wait()
        cd1.wait()
        pltpu.sync_copy(buf_d, gd_hbm.at[pl.ds(off, CG)])
        return carry

    lax.fori_loop(0, NCHG, body, 0)


def _mlp_body(gs, gd, dist8, w1e, b1e, w2e, b2e,
              w1na, w1nb, w1ne, b1n, w2n, b2n,
              w1ca, w1cb, w1ce, b1c, w2ct,
              m_out, wt_out):
    bf = jnp.bfloat16
    f32 = jnp.float32
    nsub = EB // 128
    sel = (lax.broadcasted_iota(jnp.int32, (EB, nsub), 0) // 128
           == lax.broadcasted_iota(jnp.int32, (EB, nsub), 1)).astype(f32)
    xrows = jnp.dot(sel, dist8[...], preferred_element_type=f32)
    lmask = (lax.broadcasted_iota(jnp.int32, (EB, 128), 1)
             == lax.broadcasted_iota(jnp.int32, (EB, 128), 0) % 128)
    d = jnp.sum(jnp.where(lmask, xrows, 0.0), axis=1, keepdims=True)

    a1 = d * w1e[...] + b1e[...]
    a1 = a1 * jax.nn.sigmoid(a1)
    attr = jnp.dot(a1, w2e[...], preferred_element_type=f32) + b2e[...]
    hs = gs[...].astype(bf)
    hd = gd[...].astype(bf)
    pre_n = (jnp.dot(hs, w1na[...].astype(bf), preferred_element_type=f32)
             + jnp.dot(hd, w1nb[...].astype(bf), preferred_element_type=f32)
             + jnp.dot(attr.astype(bf), w1ne[...].astype(bf),
                       preferred_element_type=f32)
             + b1n[...])
    hid_n = pre_n * jax.nn.sigmoid(pre_n)
    m_out[...] = jnp.dot(hid_n.astype(bf), w2n[...].astype(bf),
                         preferred_element_type=f32) + b2n[...]
    pre_c = (jnp.dot(hs, w1ca[...].astype(bf), preferred_element_type=f32)
             + jnp.dot(hd, w1cb[...].astype(bf), preferred_element_type=f32)
             + jnp.dot(attr.astype(bf), w1ce[...].astype(bf),
                       preferred_element_type=f32)
             + b1c[...])
    hid_c = pre_c * jax.nn.sigmoid(pre_c)
    wrow = lax.dot_general(w2ct[...], hid_c,
                           (((1,), (1,)), ((), ())),
                           preferred_element_type=f32)
    wt_out[...] = wrow.reshape(1, 1, EB)


def _full(shape):
    return pl.BlockSpec(shape, lambda i: (0, 0))


_mlp_call = pl.pallas_call(
    _mlp_body,
    grid=(GRID,),
    in_specs=[
        pl.BlockSpec((EB, D), lambda i: (i, 0)),
        pl.BlockSpec((EB, D), lambda i: (i, 0)),
        pl.BlockSpec((EB // 128, 128), lambda i: (i, 0)),
        _full((1, ED)), _full((1, ED)), _full((ED, ED)), _full((1, ED)),
        _full((D, H)), _full((D, H)), _full((ED, H)), _full((1, H)),
        _full((H, D)), _full((1, D)),
        _full((D, H)), _full((D, H)), _full((ED, H)), _full((1, H)),
        _full((1, H)),
    ],
    out_specs=[
        pl.BlockSpec((EB, D), lambda i: (i, 0)),
        pl.BlockSpec((1, 1, EB), lambda i: (i, 0, 0)),
    ],
    out_shape=[
        jax.ShapeDtypeStruct((EPAD, D), jnp.float32),
        jax.ShapeDtypeStruct((GRID, 1, EB), jnp.float32),
    ],
)


@functools.partial(
    pl.kernel,
    out_type=jax.ShapeDtypeStruct((NC, N, D), jnp.float32),
    mesh=_mesh,
    compiler_params=pltpu.CompilerParams(use_tc_tiling_on_sc=False),
    scratch_types=[
        pltpu.VMEM((C,), jnp.int32),
        pltpu.VMEM((C, D), jnp.float32),
        pltpu.VMEM_SHARED((N, D), jnp.float32),
    ],
)
def _scatter_m_k(m_hbm, dst_hbm, z128_hbm, hp_hbm, ixd0, mb0, h_acc):
    cid = lax.axis_index("c")
    sid = lax.axis_index("s")
    wid = sid * NC + cid
    r0 = sid * ROWS_PER_TILE
    pltpu.sync_copy(z128_hbm.at[pl.ds(r0, ROWS_PER_TILE)],
                    h_acc.at[pl.ds(r0, ROWS_PER_TILE)])
    plsc.subcore_barrier()
    base = wid * EW

    def body(i, carry):
        off = base + i * C
        pltpu.sync_copy(dst_hbm.at[pl.ds(off, C)], ixd0)
        pltpu.sync_copy(m_hbm.at[pl.ds(off, C)], mb0)
        pltpu.sync_copy(mb0, h_acc.at[ixd0], add=True)
        return carry

    lax.fori_loop(0, NCHUNK, body, 0)
    plsc.subcore_barrier()
    pltpu.sync_copy(h_acc.at[pl.ds(r0, ROWS_PER_TILE)],
                    hp_hbm.at[cid, pl.ds(r0, ROWS_PER_TILE)])


C2 = 400
NCHUNK2 = EW // C2


@functools.partial(
    pl.kernel,
    out_type=jax.ShapeDtypeStruct((NW, N * 4), jnp.float32),
    mesh=_mesh,
    compiler_params=pltpu.CompilerParams(use_tc_tiling_on_sc=False,
                                         needs_layout_passes=False),
    scratch_types=[
        pltpu.VMEM((C2,), jnp.int32),
        pltpu.VMEM((C2,), jnp.int32),
        pltpu.VMEM((C2,), jnp.float32),
        pltpu.VMEM((N * 4,), jnp.float32),
        pltpu.VMEM((N * 4,), jnp.float32),
    ],
)
def _scatter_x_k(w_hbm, src_hbm, dst_hbm, x4_hbm, z4_hbm, xp_hbm,
                 idx_s, idx_d, w_buf, xtab, xacc):
    cid = lax.axis_index("c")
    sid = lax.axis_index("s")
    wid = sid * NC + cid
    pltpu.sync_copy(x4_hbm, xtab)
    pltpu.sync_copy(z4_hbm, xacc)
    base = wid * EW
    lane = lax.iota(jnp.int32, 16)

    def body(i, carry):
        off = base + i * C2
        pltpu.sync_copy(src_hbm.at[pl.ds(off, C2)], idx_s)
        pltpu.sync_copy(dst_hbm.at[pl.ds(off, C2)], idx_d)
        pltpu.sync_copy(w_hbm.at[pl.ds(off, C2)], w_buf)
        for j in range(C2 // 16):
            s16 = idx_s[pl.ds(j * 16, 16)] * 4
            d16 = idx_d[pl.ds(j * 16, 16)] * 4
            w16 = w_buf[pl.ds(j * 16, 16)]
            for comp in range(3):
                xs = plsc.load_gather(xtab, [s16 + comp])
                xd = plsc.load_gather(xtab, [d16 + comp])
                plsc.addupdate_scatter(xacc, [d16 + comp], w16 * (xs - xd))
        return carry

    lax.fori_loop(0, NCHUNK2, body, 0)
    pltpu.sync_copy(xacc, xp_hbm.at[wid])


def kernel(h, x, edge_index, edge_dist,
           W1e, b1e, W2e, b2e, W1n, b1n, W2n, b2n, W1c, b1c, W2c):
    src = edge_index[0]
    dst = edge_index[1]
    gs, gd = _gather_k(h, src, dst)
    dist_pad = jnp.pad(edge_dist, (0, EPAD - E)).reshape(EPAD // 128, 128)
    m, wt = _mlp_call(
        gs, gd, dist_pad,
        W1e, b1e.reshape(1, ED), W2e, b2e.reshape(1, ED),
        W1n[:D], W1n[D:2 * D], W1n[2 * D:], b1n.reshape(1, H),
        W2n, b2n.reshape(1, D),
        W1c[:D], W1c[D:2 * D], W1c[2 * D:], b1c.reshape(1, H),
        W2c.reshape(1, H),
    )
    w1d = wt.reshape(EPAD)
    x4 = jnp.pad(x, ((0, 0), (0, 1))).reshape(N * 4)
    z128 = jnp.zeros((N, D), jnp.float32)
    z4 = jnp.zeros((N * 4,), jnp.float32)
    hp = _scatter_m_k(m, dst, z128)
    xp = _scatter_x_k(w1d, src, dst, x4, z4)
    h_out = h + hp[0] + hp[1]
    x_out = x + jnp.sum(xp, axis=0).reshape(N, 4)[:, :3]
    return (h_out, x_out)
